# initial kernel scaffold (unmeasured)
import jax
import jax.numpy as jnp
from jax import lax
from jax.experimental import pallas as pl
from jax.experimental.pallas import tpu as pltpu

N_DEV = 4


def kernel(x, w_mat):
    m_per, k = x.shape
    _, n_per = w_mat.shape

    def body(x_ref, w_ref, out_ref, comm_ref, send_sems, recv_sems):
        my_pos = lax.axis_index("i")
        left = (my_pos - 1) % N_DEV
        right = (my_pos + 1) % N_DEV

        barrier_sem = pltpu.get_barrier_semaphore()
        for nbr in [left, right]:
            pl.semaphore_signal(
                barrier_sem, inc=1,
                device_id=(nbr,), device_id_type=pl.DeviceIdType.MESH,
            )
        pl.semaphore_wait(barrier_sem, 2)

        comm_ref[0] = x_ref[...]
        out_ref[pl.ds(my_pos * m_per, m_per), :] = jnp.dot(
            x_ref[...], w_ref[...], preferred_element_type=jnp.float32
        )

        for h in range(N_DEV - 1):
            send_slot = h % 2
            recv_slot = (h + 1) % 2
            rdma = pltpu.make_async_remote_copy(
                src_ref=comm_ref.at[send_slot],
                dst_ref=comm_ref.at[recv_slot],
                send_sem=send_sems.at[send_slot],
                recv_sem=recv_sems.at[recv_slot],
                device_id=(right,),
                device_id_type=pl.DeviceIdType.MESH,
            )
            rdma.start()
            rdma.wait()

            origin = (my_pos - h - 1) % N_DEV
            out_ref[pl.ds(origin * m_per, m_per), :] = jnp.dot(
                comm_ref[recv_slot], w_ref[...],
                preferred_element_type=jnp.float32,
            )

    return pl.pallas_call(
        body,
        out_shape=jax.ShapeDtypeStruct((N_DEV * m_per, n_per), jnp.float32),
        in_specs=[
            pl.BlockSpec(memory_space=pltpu.VMEM),
            pl.BlockSpec(memory_space=pltpu.VMEM),
        ],
        out_specs=pl.BlockSpec(memory_space=pltpu.VMEM),
        scratch_shapes=[
            pltpu.VMEM((2, m_per, k), jnp.float32),
            pltpu.SemaphoreType.DMA((2,)),
            pltpu.SemaphoreType.DMA((2,)),
        ],
        compiler_params=pltpu.CompilerParams(collective_id=0),
    )(x, w_mat)


# baseline (device time: 709784 ns/iter reference)
import jax
import jax.numpy as jnp
from jax import lax
from jax.experimental import pallas as pl
from jax.experimental.pallas import tpu as pltpu

N_DEV = 4


def _all_gather(x):
    m_per, k = x.shape

    def body(x_ref, g_ref, local_sem, send_sems, recv_sems):
        my_pos = lax.axis_index("i")
        left = (my_pos - 1) % N_DEV
        right = (my_pos + 1) % N_DEV

        barrier_sem = pltpu.get_barrier_semaphore()
        for nbr in [left, right]:
            pl.semaphore_signal(
                barrier_sem, inc=1,
                device_id=(nbr,), device_id_type=pl.DeviceIdType.MESH,
            )
        pl.semaphore_wait(barrier_sem, 2)

        local_copy = pltpu.make_async_copy(
            x_ref, g_ref.at[pl.ds(my_pos * m_per, m_per)], local_sem
        )
        local_copy.start()

        for h in range(N_DEV - 1):
            o_send = (my_pos - h) % N_DEV
            src = x_ref if h == 0 else g_ref.at[pl.ds(o_send * m_per, m_per)]
            rdma = pltpu.make_async_remote_copy(
                src_ref=src,
                dst_ref=g_ref.at[pl.ds(o_send * m_per, m_per)],
                send_sem=send_sems.at[h],
                recv_sem=recv_sems.at[h],
                device_id=(right,),
                device_id_type=pl.DeviceIdType.MESH,
            )
            rdma.start()
            rdma.wait()

        local_copy.wait()

    return pl.pallas_call(
        body,
        out_shape=jax.ShapeDtypeStruct((N_DEV * m_per, k), x.dtype),
        in_specs=[pl.BlockSpec(memory_space=pl.ANY)],
        out_specs=pl.BlockSpec(memory_space=pl.ANY),
        scratch_shapes=[
            pltpu.SemaphoreType.DMA,
            pltpu.SemaphoreType.DMA((N_DEV - 1,)),
            pltpu.SemaphoreType.DMA((N_DEV - 1,)),
        ],
        compiler_params=pltpu.CompilerParams(collective_id=0),
    )(x)


def _gemm(x_full, w):
    m, k = x_full.shape
    _, n = w.shape
    bm, bn, bk = 512, 1024, 1024
    grid = (m // bm, n // bn, k // bk)

    def mm(x_ref, w_ref, o_ref, acc_ref):
        @pl.when(pl.program_id(2) == 0)
        def _():
            acc_ref[...] = jnp.zeros_like(acc_ref)

        acc_ref[...] += jnp.dot(
            x_ref[...], w_ref[...], preferred_element_type=jnp.float32
        )

        @pl.when(pl.program_id(2) == grid[2] - 1)
        def _():
            o_ref[...] = acc_ref[...]

    return pl.pallas_call(
        mm,
        grid=grid,
        in_specs=[
            pl.BlockSpec((bm, bk), lambda i, j, kk: (i, kk)),
            pl.BlockSpec((bk, bn), lambda i, j, kk: (kk, j)),
        ],
        out_specs=pl.BlockSpec((bm, bn), lambda i, j, kk: (i, j)),
        out_shape=jax.ShapeDtypeStruct((m, n), jnp.float32),
        scratch_shapes=[pltpu.VMEM((bm, bn), jnp.float32)],
        compiler_params=pltpu.CompilerParams(
            dimension_semantics=("parallel", "parallel", "arbitrary")
        ),
    )(x_full, w)


def kernel(x, w_mat):
    x_full = _all_gather(x)
    return _gemm(x_full, w_mat)


# device time: 342681 ns/iter; 2.0713x vs baseline; 2.0713x over previous
import jax
import jax.numpy as jnp
from jax import lax
from jax.experimental import pallas as pl
from jax.experimental.pallas import tpu as pltpu

N_DEV = 4


def kernel(x, w_mat):
    m_per, k = x.shape
    _, n_per = w_mat.shape
    m_half = m_per // 2
    bm, bn = 512, 512

    def body(x_ref, w_ref, out_ref, g_ref,
             cw_send, cw_recv, ccw_send, ccw_recv):
        my_pos = lax.axis_index("i")
        left = (my_pos - 1) % N_DEV
        right = (my_pos + 1) % N_DEV

        barrier_sem = pltpu.get_barrier_semaphore()
        for nbr in [left, right]:
            pl.semaphore_signal(
                barrier_sem, inc=1,
                device_id=(nbr,), device_id_type=pl.DeviceIdType.MESH,
            )
        pl.semaphore_wait(barrier_sem, 2)

        def mm(xt, wt, ot):
            ot[...] = jnp.dot(
                xt[...], wt[...], preferred_element_type=jnp.float32
            )

        def do_gemm(src_ref, rows, out_row0):
            pipe = pltpu.emit_pipeline(
                mm,
                grid=(rows // bm, n_per // bn),
                in_specs=[
                    pl.BlockSpec((bm, k), lambda i, j: (i, 0)),
                    pl.BlockSpec((k, bn), lambda i, j: (0, j)),
                ],
                out_specs=[pl.BlockSpec((bm, bn), lambda i, j: (i, j))],
            )
            pipe(src_ref, w_ref, out_ref.at[pl.ds(out_row0, rows)])

        def make_cw(h):
            o = (my_pos - h) % N_DEV
            src = (x_ref.at[pl.ds(0, m_half)] if h == 0
                   else g_ref.at[pl.ds(o * m_per, m_half)])
            return pltpu.make_async_remote_copy(
                src_ref=src,
                dst_ref=g_ref.at[pl.ds(o * m_per, m_half)],
                send_sem=cw_send.at[h],
                recv_sem=cw_recv.at[h],
                device_id=(right,),
                device_id_type=pl.DeviceIdType.MESH,
            )

        def make_ccw(h):
            o = (my_pos + h) % N_DEV
            src = (x_ref.at[pl.ds(m_half, m_half)] if h == 0
                   else g_ref.at[pl.ds(o * m_per + m_half, m_half)])
            return pltpu.make_async_remote_copy(
                src_ref=src,
                dst_ref=g_ref.at[pl.ds(o * m_per + m_half, m_half)],
                send_sem=ccw_send.at[h],
                recv_sem=ccw_recv.at[h],
                device_id=(left,),
                device_id_type=pl.DeviceIdType.MESH,
            )

        cw = make_cw(0)
        ccw = make_ccw(0)
        cw.start()
        ccw.start()

        do_gemm(x_ref, m_per, my_pos * m_per)

        for h in range(N_DEV - 1):
            cw.wait()
            ccw.wait()
            o_top = (my_pos - h - 1) % N_DEV
            o_bot = (my_pos + h + 1) % N_DEV
            if h + 1 < N_DEV - 1:
                cw = make_cw(h + 1)
                ccw = make_ccw(h + 1)
                cw.start()
                ccw.start()
            do_gemm(g_ref.at[pl.ds(o_top * m_per, m_half)],
                    m_half, o_top * m_per)
            do_gemm(g_ref.at[pl.ds(o_bot * m_per + m_half, m_half)],
                    m_half, o_bot * m_per + m_half)

    out, _ = pl.pallas_call(
        body,
        out_shape=(
            jax.ShapeDtypeStruct((N_DEV * m_per, n_per), jnp.float32),
            jax.ShapeDtypeStruct((N_DEV * m_per, k), jnp.float32),
        ),
        in_specs=[
            pl.BlockSpec(memory_space=pl.ANY),
            pl.BlockSpec(memory_space=pl.ANY),
        ],
        out_specs=(
            pl.BlockSpec(memory_space=pl.ANY),
            pl.BlockSpec(memory_space=pl.ANY),
        ),
        scratch_shapes=[
            pltpu.SemaphoreType.DMA((N_DEV - 1,)),
            pltpu.SemaphoreType.DMA((N_DEV - 1,)),
            pltpu.SemaphoreType.DMA((N_DEV - 1,)),
            pltpu.SemaphoreType.DMA((N_DEV - 1,)),
        ],
        compiler_params=pltpu.CompilerParams(
            collective_id=0,
            vmem_limit_bytes=56 * 1024 * 1024,
        ),
    )(x, w_mat)
    return out


# device time: 309201 ns/iter; 2.2955x vs baseline; 1.1083x over previous
import jax
import jax.numpy as jnp
from jax import lax
from jax.experimental import pallas as pl
from jax.experimental.pallas import tpu as pltpu

N_DEV = 4
N_SEG = 2


def kernel(x, w_mat):
    m_per, k = x.shape
    _, n_per = w_mat.shape
    m_half = m_per // 2
    m_seg = m_half // N_SEG

    f_seg = m_seg // 2
    n_fine = 2 * N_SEG

    def body(x_ref, w_ref, out_ref, g_ref,
             w_vmem, x_vmem, o_vmem, xf_vmem, of_vmem,
             w_sem, in_sem, out_sem,
             cw_send, cw_recv, ccw_send, ccw_recv,
             fcw_send, fcw_recv, fccw_send, fccw_recv):
        my_pos = lax.axis_index("i")
        left = (my_pos - 1) % N_DEV
        right = (my_pos + 1) % N_DEV

        barrier_sem = pltpu.get_barrier_semaphore()
        for nbr in [left, right]:
            pl.semaphore_signal(
                barrier_sem, inc=1,
                device_id=(nbr,), device_id_type=pl.DeviceIdType.MESH,
            )
        pl.semaphore_wait(barrier_sem, 2)

        def cw_rows(h, s):
            return ((my_pos - h) % N_DEV) * m_per + s * m_seg

        def ccw_rows(h, s):
            return ((my_pos + h) % N_DEV) * m_per + m_half + s * m_seg

        def make_cw(h, s):
            r = cw_rows(h, s)
            src = (x_ref.at[pl.ds(s * m_seg, m_seg)] if h == 0
                   else g_ref.at[pl.ds(r, m_seg)])
            return pltpu.make_async_remote_copy(
                src_ref=src,
                dst_ref=g_ref.at[pl.ds(r, m_seg)],
                send_sem=cw_send.at[h, s],
                recv_sem=cw_recv.at[h, s],
                device_id=(right,),
                device_id_type=pl.DeviceIdType.MESH,
            )

        def make_ccw(h, s):
            r = ccw_rows(h, s)
            src = (x_ref.at[pl.ds(m_half + s * m_seg, m_seg)] if h == 0
                   else g_ref.at[pl.ds(r, m_seg)])
            return pltpu.make_async_remote_copy(
                src_ref=src,
                dst_ref=g_ref.at[pl.ds(r, m_seg)],
                send_sem=ccw_send.at[h, s],
                recv_sem=ccw_recv.at[h, s],
                device_id=(left,),
                device_id_type=pl.DeviceIdType.MESH,
            )

        w_copy = pltpu.make_async_copy(w_ref, w_vmem, w_sem)
        w_copy.start()
        for s in range(N_SEG):
            make_cw(0, s).start()
            make_ccw(0, s).start()
        w_copy.wait()

        def gemm_rows(src_ref, src_row0, out_row0):
            cp_in = pltpu.make_async_copy(
                src_ref.at[pl.ds(src_row0, m_seg)], x_vmem, in_sem
            )
            cp_in.start()
            cp_in.wait()
            o_vmem[...] = jnp.dot(
                x_vmem[...], w_vmem[...], preferred_element_type=jnp.float32
            )
            cp_out = pltpu.make_async_copy(
                o_vmem, out_ref.at[pl.ds(out_row0, m_seg)], out_sem
            )
            cp_out.start()
            cp_out.wait()

        def local_body(j, c):
            gemm_rows(x_ref, j * m_seg, my_pos * m_per + j * m_seg)
            return c

        lax.fori_loop(0, 2 * N_SEG, local_body, 0)

        def recv_batch(h):
            def batch_body(t, c):
                row = jnp.where(
                    t < N_SEG,
                    cw_rows(h + 1, t),
                    ccw_rows(h + 1, t - N_SEG),
                )
                gemm_rows(g_ref, row, row)
                return c
            lax.fori_loop(0, 2 * N_SEG, batch_body, 0)

        h_last = N_DEV - 2

        def make_fine(q, rows_fn, send_sems, recv_sems, target):
            r = rows_fn(h_last, 0) + q * f_seg
            return pltpu.make_async_remote_copy(
                src_ref=g_ref.at[pl.ds(r, f_seg)],
                dst_ref=g_ref.at[pl.ds(r, f_seg)],
                send_sem=send_sems.at[q],
                recv_sem=recv_sems.at[q],
                device_id=(target,),
                device_id_type=pl.DeviceIdType.MESH,
            )

        for h in range(N_DEV - 2):
            for s in range(N_SEG):
                make_cw(h, s).wait()
                make_ccw(h, s).wait()
                if h + 1 < N_DEV - 2:
                    make_cw(h + 1, s).start()
                    make_ccw(h + 1, s).start()
                else:
                    for sub in range(2):
                        q = 2 * s + sub
                        make_fine(q, cw_rows, fcw_send, fcw_recv,
                                  right).start()
                        make_fine(q, ccw_rows, fccw_send, fccw_recv,
                                  left).start()
            recv_batch(h)

        top_base = ((my_pos + 1) % N_DEV) * m_per
        bot_base = ((my_pos - 1) % N_DEV) * m_per + m_half

        def gemm_fine(row0):
            cp_in = pltpu.make_async_copy(
                g_ref.at[pl.ds(row0, f_seg)], xf_vmem, in_sem
            )
            cp_in.start()
            cp_in.wait()
            of_vmem[...] = jnp.dot(
                xf_vmem[...], w_vmem[...], preferred_element_type=jnp.float32
            )
            cp_out = pltpu.make_async_copy(
                of_vmem, out_ref.at[pl.ds(row0, f_seg)], out_sem
            )
            cp_out.start()
            cp_out.wait()

        for q in range(n_fine):
            make_fine(q, cw_rows, fcw_send, fcw_recv, right).wait()
            make_fine(q, ccw_rows, fccw_send, fccw_recv, left).wait()

            def fine_body(t, c, q=q):
                row = jnp.where(
                    t < 1, top_base + q * f_seg, bot_base + q * f_seg
                )
                gemm_fine(row)
                return c

            lax.fori_loop(0, 2, fine_body, 0)

    out, _ = pl.pallas_call(
        body,
        out_shape=(
            jax.ShapeDtypeStruct((N_DEV * m_per, n_per), jnp.float32),
            jax.ShapeDtypeStruct((N_DEV * m_per, k), jnp.float32),
        ),
        in_specs=[
            pl.BlockSpec(memory_space=pl.ANY),
            pl.BlockSpec(memory_space=pl.ANY),
        ],
        out_specs=(
            pl.BlockSpec(memory_space=pl.ANY),
            pl.BlockSpec(memory_space=pl.ANY),
        ),
        scratch_shapes=[
            pltpu.VMEM((k, n_per), jnp.float32),
            pltpu.VMEM((m_seg, k), jnp.float32),
            pltpu.VMEM((m_seg, n_per), jnp.float32),
            pltpu.VMEM((f_seg, k), jnp.float32),
            pltpu.VMEM((f_seg, n_per), jnp.float32),
            pltpu.SemaphoreType.DMA,
            pltpu.SemaphoreType.DMA,
            pltpu.SemaphoreType.DMA,
            pltpu.SemaphoreType.DMA((N_DEV - 1, N_SEG)),
            pltpu.SemaphoreType.DMA((N_DEV - 1, N_SEG)),
            pltpu.SemaphoreType.DMA((N_DEV - 1, N_SEG)),
            pltpu.SemaphoreType.DMA((N_DEV - 1, N_SEG)),
            pltpu.SemaphoreType.DMA((n_fine,)),
            pltpu.SemaphoreType.DMA((n_fine,)),
            pltpu.SemaphoreType.DMA((n_fine,)),
            pltpu.SemaphoreType.DMA((n_fine,)),
        ],
        compiler_params=pltpu.CompilerParams(
            collective_id=0,
            vmem_limit_bytes=56 * 1024 * 1024,
        ),
    )(x, w_mat)
    return out
